# baseline (speedup 1.0000x reference)
import functools

import jax
import jax.numpy as jnp
from jax import lax
from jax.experimental import pallas as pl
from jax.experimental.pallas import tpu as pltpu
from jax.experimental.pallas import tpu_sc as plsc

NC = 2
NS = 16
NW = NC * NS
QB = 4096


def _build(A, V, D, B):
    R = A * D
    assert R % NW == 0
    rpw = R // NW
    assert D == NW
    nq = B // QB
    assert B % QB == 0 and V % 2 == 0

    mesh = plsc.VectorSubcoreMesh(
        core_axis_name="c", subcore_axis_name="s",
        num_cores=NC, num_subcores=NS)

    @functools.partial(
        pl.kernel,
        out_type=[
            jax.ShapeDtypeStruct((R, B), jnp.float32),
            jax.ShapeDtypeStruct((D, B), jnp.float32),
        ],
        mesh=mesh,
        compiler_params=pltpu.CompilerParams(
            use_tc_tiling_on_sc=True, needs_layout_passes=False),
        scratch_types=[
            pltpu.VMEM((V,), jnp.float32),
            pltpu.VMEM((B,), jnp.int32),
            pltpu.VMEM((2, QB), jnp.float32),
            pltpu.SemaphoreType.DMA,
            pltpu.SemaphoreType.DMA,
            pltpu.SemaphoreType.DMA,
            pltpu.SemaphoreType.DMA,
            pltpu.SemaphoreType.DMA,
        ],
    )
    def lookup(mt_hbm, ma_hbm, tab_hbm, atab_hbm, out_t_hbm, out_a_hbm,
               row_v, idx_v, val_v, sem_i, sem_r0, sem_r1, sem_o0, sem_o1):
        wid = lax.axis_index("s") * NC + lax.axis_index("c")
        H = V // 2

        def gather_quarter(q, vb):
            def g(i, _):
                s0 = pl.ds(q * QB + i * 32, 16)
                s1 = pl.ds(q * QB + i * 32 + 16, 16)
                val_v[vb, pl.ds(i * 32, 16)] = plsc.load_gather(
                    row_v, [idx_v[s0]])
                val_v[vb, pl.ds(i * 32 + 16, 16)] = plsc.load_gather(
                    row_v, [idx_v[s1]])
                return 0
            lax.fori_loop(0, QB // 32, g, 0)

        def process_row(out_ref, orow):
            for q in range(nq):
                vb = q % 2
                if q >= 2:
                    pltpu.make_async_copy(
                        val_v.at[vb],
                        out_ref.at[orow, pl.ds((q - 2) * QB, QB)],
                        sem_o0 if vb == 0 else sem_o1).wait()
                gather_quarter(q, vb)
                pltpu.make_async_copy(
                    val_v.at[vb], out_ref.at[orow, pl.ds(q * QB, QB)],
                    sem_o0 if vb == 0 else sem_o1).start()
            pltpu.make_async_copy(
                val_v.at[0], out_ref.at[orow, pl.ds((nq - 2) * QB, QB)],
                sem_o0).wait()
            pltpu.make_async_copy(
                val_v.at[1], out_ref.at[orow, pl.ds((nq - 1) * QB, QB)],
                sem_o1).wait()

        def start_row_stream(src_row):
            pltpu.make_async_copy(src_row, row_v, sem_r0).start()

        def wait_row_stream(src_row):
            pltpu.make_async_copy(src_row, row_v, sem_r0).wait()

        def row_task(j, _):
            r = wid * rpw + j
            a = r // D
            src = tab_hbm.at[a, r % D]
            start_row_stream(src)
            fresh = jnp.logical_or(j == 0, r % D == 0)

            @pl.when(fresh)
            def _():
                pltpu.make_async_copy(mt_hbm.at[a], idx_v, sem_i).start()
                pltpu.make_async_copy(mt_hbm.at[a], idx_v, sem_i).wait()
            wait_row_stream(src)
            process_row(out_t_hbm, r)
            return 0
        lax.fori_loop(0, rpw, row_task, 0)

        src = atab_hbm.at[wid]
        start_row_stream(src)
        pltpu.make_async_copy(ma_hbm, idx_v, sem_i).start()
        pltpu.make_async_copy(ma_hbm, idx_v, sem_i).wait()
        wait_row_stream(src)
        process_row(out_a_hbm, wid)

    return lookup


def kernel(mask_tuple, mask_idx, mask_attrs, tables):
    B, A = mask_tuple.shape
    _, V, D = tables.shape
    lookup = _build(A, V, D, B)

    tab = tables.transpose(0, 2, 1)
    atab = lax.dynamic_index_in_dim(tab, mask_idx, 0, keepdims=False)
    mt = mask_tuple.T
    out_t, out_a = lookup(mt, mask_attrs, tab, atab)
    return out_t.T, out_a.T


# speedup vs baseline: 1.1680x; 1.1680x over previous
import functools

import jax
import jax.numpy as jnp
from jax import lax
from jax.experimental import pallas as pl
from jax.experimental.pallas import tpu as pltpu
from jax.experimental.pallas import tpu_sc as plsc

NC = 2
NS = 16
NW = NC * NS
HB = 8192


def _build(A, V, D, B):
    R = A * D
    assert R % NW == 0
    rpw = R // NW
    assert D == NW
    nh = B // HB
    assert B % HB == 0

    mesh = plsc.VectorSubcoreMesh(
        core_axis_name="c", subcore_axis_name="s",
        num_cores=NC, num_subcores=NS)

    @functools.partial(
        pl.kernel,
        out_type=[
            jax.ShapeDtypeStruct((R, B), jnp.float32),
            jax.ShapeDtypeStruct((D, B), jnp.float32),
        ],
        mesh=mesh,
        compiler_params=pltpu.CompilerParams(
            use_tc_tiling_on_sc=True, needs_layout_passes=False),
        scratch_types=[
            pltpu.VMEM((V,), jnp.float32),
            pltpu.VMEM((B,), jnp.int32),
            pltpu.VMEM((HB,), jnp.float32),
        ],
    )
    def lookup(mt_hbm, ma_hbm, tab_hbm, atab_hbm, out_t_hbm, out_a_hbm,
               row_v, idx_v, val_v):
        wid = lax.axis_index("s") * NC + lax.axis_index("c")

        def process_row(out_ref, orow):
            for h in range(nh):
                def g(i, _):
                    s0 = pl.ds(h * HB + i * 32, 16)
                    s1 = pl.ds(h * HB + i * 32 + 16, 16)
                    val_v[pl.ds(i * 32, 16)] = plsc.load_gather(
                        row_v, [idx_v[s0]])
                    val_v[pl.ds(i * 32 + 16, 16)] = plsc.load_gather(
                        row_v, [idx_v[s1]])
                    return 0
                lax.fori_loop(0, HB // 32, g, 0)
                pltpu.sync_copy(val_v, out_ref.at[orow, pl.ds(h * HB, HB)])

        def row_task(j, _):
            r = wid * rpw + j
            a = r // D
            @pl.when(jnp.logical_or(j == 0, r % D == 0))
            def _():
                pltpu.sync_copy(mt_hbm.at[a], idx_v)
            pltpu.sync_copy(tab_hbm.at[a, r % D], row_v)
            process_row(out_t_hbm, r)
            return 0
        lax.fori_loop(0, rpw, row_task, 0)

        pltpu.sync_copy(ma_hbm, idx_v)
        pltpu.sync_copy(atab_hbm.at[wid], row_v)
        process_row(out_a_hbm, wid)

    return lookup


def kernel(mask_tuple, mask_idx, mask_attrs, tables):
    B, A = mask_tuple.shape
    _, V, D = tables.shape
    lookup = _build(A, V, D, B)

    tab = tables.transpose(0, 2, 1)
    atab = lax.dynamic_index_in_dim(tab, mask_idx, 0, keepdims=False)
    mt = mask_tuple.T
    out_t, out_a = lookup(mt, mask_attrs, tab, atab)
    return out_t.T, out_a.T


# speedup vs baseline: 1.9756x; 1.6914x over previous
import functools

import jax
import jax.numpy as jnp
from jax import lax
from jax.experimental import pallas as pl
from jax.experimental.pallas import tpu as pltpu
from jax.experimental.pallas import tpu_sc as plsc

NC = 2
NS = 16
NW = NC * NS
HB = 8192


def _build(A, V, D, B):
    R = A * D
    assert R % NW == 0
    rpw = R // NW
    assert D == NW
    nh = B // HB
    assert B % HB == 0

    mesh = plsc.VectorSubcoreMesh(
        core_axis_name="c", subcore_axis_name="s",
        num_cores=NC, num_subcores=NS)

    @functools.partial(
        pl.kernel,
        out_type=[
            jax.ShapeDtypeStruct((R, B), jnp.float32),
            jax.ShapeDtypeStruct((D, B), jnp.float32),
        ],
        mesh=mesh,
        compiler_params=pltpu.CompilerParams(
            use_tc_tiling_on_sc=True, needs_layout_passes=False),
        scratch_types=[
            pltpu.VMEM((V,), jnp.float32),
            pltpu.VMEM((B,), jnp.int32),
            pltpu.VMEM((HB,), jnp.float32),
        ],
    )
    def lookup(mt_hbm, ma_hbm, tab_hbm, atab_hbm, out_t_hbm, out_a_hbm,
               row_v, idx_v, val_v):
        wid = lax.axis_index("s") * NC + lax.axis_index("c")

        def process_row(out_ref, orow):
            for h in range(nh):
                @plsc.parallel_loop(0, HB // 32, unroll=4)
                def g(i):
                    s0 = pl.ds(h * HB + i * 32, 16)
                    s1 = pl.ds(h * HB + i * 32 + 16, 16)
                    val_v[pl.ds(i * 32, 16)] = plsc.load_gather(
                        row_v, [idx_v[s0]])
                    val_v[pl.ds(i * 32 + 16, 16)] = plsc.load_gather(
                        row_v, [idx_v[s1]])
                pltpu.sync_copy(val_v, out_ref.at[orow, pl.ds(h * HB, HB)])

        def row_task(j, _):
            r = wid * rpw + j
            a = r // D
            @pl.when(jnp.logical_or(j == 0, r % D == 0))
            def _():
                pltpu.sync_copy(mt_hbm.at[a], idx_v)
            pltpu.sync_copy(tab_hbm.at[a, r % D], row_v)
            process_row(out_t_hbm, r)
            return 0
        lax.fori_loop(0, rpw, row_task, 0)

        pltpu.sync_copy(ma_hbm, idx_v)
        pltpu.sync_copy(atab_hbm.at[wid], row_v)
        process_row(out_a_hbm, wid)

    return lookup


def kernel(mask_tuple, mask_idx, mask_attrs, tables):
    B, A = mask_tuple.shape
    _, V, D = tables.shape
    lookup = _build(A, V, D, B)

    tab = tables.transpose(0, 2, 1)
    atab = lax.dynamic_index_in_dim(tab, mask_idx, 0, keepdims=False)
    mt = mask_tuple.T
    out_t, out_a = lookup(mt, mask_attrs, tab, atab)
    return out_t.T, out_a.T


# speedup vs baseline: 1.9777x; 1.0011x over previous
import functools

import jax
import jax.numpy as jnp
from jax import lax
from jax.experimental import pallas as pl
from jax.experimental.pallas import tpu as pltpu
from jax.experimental.pallas import tpu_sc as plsc

NC = 2
NS = 16
NW = NC * NS
HB = 8192


def _build(A, V, D, B):
    R = A * D
    assert R % NW == 0
    rpw = R // NW
    assert D == NW
    nh = B // HB
    assert B % HB == 0

    mesh = plsc.VectorSubcoreMesh(
        core_axis_name="c", subcore_axis_name="s",
        num_cores=NC, num_subcores=NS)

    @functools.partial(
        pl.kernel,
        out_type=[
            jax.ShapeDtypeStruct((R, B), jnp.float32),
            jax.ShapeDtypeStruct((D, B), jnp.float32),
        ],
        mesh=mesh,
        compiler_params=pltpu.CompilerParams(
            use_tc_tiling_on_sc=True, needs_layout_passes=False),
        scratch_types=[
            pltpu.VMEM((V,), jnp.float32),
            pltpu.VMEM((B,), jnp.int32),
            pltpu.VMEM((HB,), jnp.float32),
        ],
    )
    def lookup(mt_hbm, ma_hbm, tab_hbm, atab_hbm, out_t_hbm, out_a_hbm,
               row_v, idx_v, val_v):
        wid = lax.axis_index("s") * NC + lax.axis_index("c")

        def process_row(out_ref, orow):
            for h in range(nh):
                @plsc.parallel_loop(0, HB // 32, unroll=8)
                def g(i):
                    s0 = pl.ds(h * HB + i * 32, 16)
                    s1 = pl.ds(h * HB + i * 32 + 16, 16)
                    val_v[pl.ds(i * 32, 16)] = plsc.load_gather(
                        row_v, [idx_v[s0]])
                    val_v[pl.ds(i * 32 + 16, 16)] = plsc.load_gather(
                        row_v, [idx_v[s1]])
                pltpu.sync_copy(val_v, out_ref.at[orow, pl.ds(h * HB, HB)])

        def row_task(j, _):
            r = wid * rpw + j
            a = r // D
            @pl.when(jnp.logical_or(j == 0, r % D == 0))
            def _():
                pltpu.sync_copy(mt_hbm.at[a], idx_v)
            pltpu.sync_copy(tab_hbm.at[a, r % D], row_v)
            process_row(out_t_hbm, r)
            return 0
        lax.fori_loop(0, rpw, row_task, 0)

        pltpu.sync_copy(ma_hbm, idx_v)
        pltpu.sync_copy(atab_hbm.at[wid], row_v)
        process_row(out_a_hbm, wid)

    return lookup


def kernel(mask_tuple, mask_idx, mask_attrs, tables):
    B, A = mask_tuple.shape
    _, V, D = tables.shape
    lookup = _build(A, V, D, B)

    tab = tables.transpose(0, 2, 1)
    atab = lax.dynamic_index_in_dim(tab, mask_idx, 0, keepdims=False)
    mt = mask_tuple.T
    out_t, out_a = lookup(mt, mask_attrs, tab, atab)
    return out_t.T, out_a.T


# speedup vs baseline: 1.9906x; 1.0065x over previous
import functools

import jax
import jax.numpy as jnp
from jax import lax
from jax.experimental import pallas as pl
from jax.experimental.pallas import tpu as pltpu
from jax.experimental.pallas import tpu_sc as plsc

NC = 2
NS = 16
NW = NC * NS
HB = 4096


def _build(A, V, D, B):
    R = A * D
    assert R % NW == 0
    rpw = R // NW
    assert D == NW
    nh = B // HB
    assert B % HB == 0

    mesh = plsc.VectorSubcoreMesh(
        core_axis_name="c", subcore_axis_name="s",
        num_cores=NC, num_subcores=NS)

    @functools.partial(
        pl.kernel,
        out_type=[
            jax.ShapeDtypeStruct((R, B), jnp.float32),
            jax.ShapeDtypeStruct((D, B), jnp.float32),
        ],
        mesh=mesh,
        compiler_params=pltpu.CompilerParams(
            use_tc_tiling_on_sc=True, needs_layout_passes=False),
        scratch_types=[
            pltpu.VMEM((V,), jnp.float32),
            pltpu.VMEM((B,), jnp.int32),
            pltpu.VMEM((2, HB), jnp.float32),
            pltpu.SemaphoreType.DMA,
            pltpu.SemaphoreType.DMA,
        ],
    )
    def lookup(mt_hbm, ma_hbm, tab_hbm, atab_hbm, out_t_hbm, out_a_hbm,
               row_v, idx_v, val_v, sem_o0, sem_o1):
        wid = lax.axis_index("s") * NC + lax.axis_index("c")

        def out_copy(out_ref, orow, h, vb):
            return pltpu.make_async_copy(
                val_v.at[vb], out_ref.at[orow, pl.ds(h * HB, HB)],
                sem_o0 if vb == 0 else sem_o1)

        def process_row(out_ref, orow):
            for h in range(nh):
                vb = h % 2
                if h >= 2:
                    out_copy(out_ref, orow, h - 2, vb).wait()

                @plsc.parallel_loop(0, HB // 32, unroll=8)
                def g(i):
                    s0 = pl.ds(h * HB + i * 32, 16)
                    s1 = pl.ds(h * HB + i * 32 + 16, 16)
                    val_v[vb, pl.ds(i * 32, 16)] = plsc.load_gather(
                        row_v, [idx_v[s0]])
                    val_v[vb, pl.ds(i * 32 + 16, 16)] = plsc.load_gather(
                        row_v, [idx_v[s1]])
                out_copy(out_ref, orow, h, vb).start()
            out_copy(out_ref, orow, nh - 2, (nh - 2) % 2).wait()
            out_copy(out_ref, orow, nh - 1, (nh - 1) % 2).wait()

        def row_task(j, _):
            r = wid * rpw + j
            a = r // D
            @pl.when(jnp.logical_or(j == 0, r % D == 0))
            def _():
                pltpu.sync_copy(mt_hbm.at[a], idx_v)
            pltpu.sync_copy(tab_hbm.at[a, r % D], row_v)
            process_row(out_t_hbm, r)
            return 0
        lax.fori_loop(0, rpw, row_task, 0)

        pltpu.sync_copy(ma_hbm, idx_v)
        pltpu.sync_copy(atab_hbm.at[wid], row_v)
        process_row(out_a_hbm, wid)

    return lookup


def kernel(mask_tuple, mask_idx, mask_attrs, tables):
    B, A = mask_tuple.shape
    _, V, D = tables.shape
    lookup = _build(A, V, D, B)

    tab = tables.transpose(0, 2, 1)
    atab = lax.dynamic_index_in_dim(tab, mask_idx, 0, keepdims=False)
    mt = mask_tuple.T
    out_t, out_a = lookup(mt, mask_attrs, tab, atab)
    return out_t.T, out_a.T


# speedup vs baseline: 2.0552x; 1.0325x over previous
import functools

import jax
import jax.numpy as jnp
from jax import lax
from jax.experimental import pallas as pl
from jax.experimental.pallas import tpu as pltpu
from jax.experimental.pallas import tpu_sc as plsc

NC = 2
NS = 16
NW = NC * NS
HB = 4096


def _build(A, V, D, B):
    R = A * D
    assert R % NW == 0
    rpw = R // NW
    assert D == NW
    nh = B // HB
    assert B % HB == 0

    mesh = plsc.VectorSubcoreMesh(
        core_axis_name="c", subcore_axis_name="s",
        num_cores=NC, num_subcores=NS)

    @functools.partial(
        pl.kernel,
        out_type=[
            jax.ShapeDtypeStruct((R, B), jnp.float32),
            jax.ShapeDtypeStruct((D, B), jnp.float32),
        ],
        mesh=mesh,
        compiler_params=pltpu.CompilerParams(
            use_tc_tiling_on_sc=True, needs_layout_passes=False),
        scratch_types=[
            pltpu.VMEM((V,), jnp.float32),
            pltpu.VMEM((B,), jnp.int32),
            pltpu.VMEM((2, HB), jnp.float32),
            pltpu.VMEM((16,), jnp.int32),
            pltpu.SemaphoreType.DMA,
            pltpu.SemaphoreType.DMA,
        ],
    )
    def lookup(mt_hbm, ma_hbm, tab_hbm, mi_hbm, out_t_hbm, out_a_hbm,
               row_v, idx_v, val_v, mi_v, sem_o0, sem_o1):
        wid = lax.axis_index("s") * NC + lax.axis_index("c")

        def out_copy(out_ref, orow, h, vb):
            return pltpu.make_async_copy(
                val_v.at[vb], out_ref.at[orow, pl.ds(h * HB, HB)],
                sem_o0 if vb == 0 else sem_o1)

        def process_row(out_ref, orow):
            for h in range(nh):
                vb = h % 2
                if h >= 2:
                    out_copy(out_ref, orow, h - 2, vb).wait()

                @plsc.parallel_loop(0, HB // 32, unroll=8)
                def g(i):
                    s0 = pl.ds(h * HB + i * 32, 16)
                    s1 = pl.ds(h * HB + i * 32 + 16, 16)
                    val_v[vb, pl.ds(i * 32, 16)] = plsc.load_gather(
                        row_v, [idx_v[s0]])
                    val_v[vb, pl.ds(i * 32 + 16, 16)] = plsc.load_gather(
                        row_v, [idx_v[s1]])
                out_copy(out_ref, orow, h, vb).start()
            out_copy(out_ref, orow, nh - 2, (nh - 2) % 2).wait()
            out_copy(out_ref, orow, nh - 1, (nh - 1) % 2).wait()

        def row_task(j, _):
            r = wid * rpw + j
            a = r // D
            @pl.when(jnp.logical_or(j == 0, r % D == 0))
            def _():
                pltpu.sync_copy(mt_hbm.at[a], idx_v)
            pltpu.sync_copy(tab_hbm.at[a, r % D], row_v)
            process_row(out_t_hbm, r)
            return 0
        lax.fori_loop(0, rpw, row_task, 0)

        pltpu.sync_copy(mi_hbm, mi_v)
        pltpu.sync_copy(ma_hbm, idx_v)
        mi = mi_v[pl.ds(0, 16)][0]
        pltpu.sync_copy(tab_hbm.at[mi, wid], row_v)
        process_row(out_a_hbm, wid)

    return lookup


def kernel(mask_tuple, mask_idx, mask_attrs, tables):
    B, A = mask_tuple.shape
    _, V, D = tables.shape
    lookup = _build(A, V, D, B)

    tab = tables.transpose(0, 2, 1)
    mi = jnp.full((16,), mask_idx, dtype=jnp.int32)
    mt = mask_tuple.T
    out_t, out_a = lookup(mt, mask_attrs, tab, mi)
    return out_t.T, out_a.T
